# Initial kernel scaffold; baseline (speedup 1.0000x reference)
import jax
import jax.numpy as jnp
from jax.experimental import pallas as pl


def kernel(input_ids, token_type_ids, position_ids, word_table, pos_table, type_table, W, b, gamma, beta):
    raise NotImplementedError("write your pallas kernel here")


# speedup vs baseline: 5.1384x; 5.1384x over previous
import functools

import jax
import jax.numpy as jnp
from jax import lax
from jax.experimental import pallas as pl
from jax.experimental.pallas import tpu as pltpu
from jax.experimental.pallas import tpu_sc as plsc

_NC, _NS = 2, 16
_NW = _NC * _NS


def _sc_gather(table, ids):
    V, E = table.shape
    N = ids.shape[0]
    per_w = N // _NW
    CH = 512
    n_ch = per_w // CH
    mesh = plsc.VectorSubcoreMesh(core_axis_name="c", subcore_axis_name="s")

    @functools.partial(
        pl.kernel,
        out_type=jax.ShapeDtypeStruct((N, E), jnp.float32),
        mesh=mesh,
        scratch_types=[
            pltpu.VMEM((per_w,), jnp.int32),
            pltpu.VMEM((CH, E), jnp.float32),
            pltpu.SemaphoreType.DMA,
        ],
    )
    def k(table_hbm, idx_hbm, out_hbm, idx_v, rows_v, sem):
        wid = lax.axis_index("s") * _NC + lax.axis_index("c")
        base = wid * per_w
        pltpu.sync_copy(idx_hbm.at[pl.ds(base, per_w)], idx_v)

        @pl.loop(0, n_ch)
        def _(i):
            start = i * CH
            pltpu.async_copy(
                table_hbm.at[idx_v.at[pl.ds(start, CH)]], rows_v, sem
            ).wait()
            pltpu.sync_copy(rows_v, out_hbm.at[pl.ds(base + start, CH)])

    return k(table, ids)


def _tc_body(we_ref, ttf_ref, w_ref, csum_ref, dg_ref, out_ref):
    BB, S, E = we_ref.shape
    H = out_ref.shape[2]
    we2 = we_ref[...].reshape(BB * S, E)
    zrow = jnp.zeros((1, E), jnp.float32)
    left = jnp.concatenate([we2[1:], zrow], axis=0)
    right = jnp.concatenate([zrow, we2[:-1]], axis=0)
    r = lax.broadcasted_iota(jnp.int32, (BB * S, 1), 0) % S
    left = jnp.where(r == (S - 1), 0.0, left)
    right = jnp.where(r == 0, 0.0, right)
    x = jnp.dot(we2, w_ref[E:2 * E], preferred_element_type=jnp.float32)
    x = x + jnp.dot(left, w_ref[:E], preferred_element_type=jnp.float32)
    x = x + jnp.dot(right, w_ref[2 * E:], preferred_element_type=jnp.float32)
    x3 = x.reshape(BB, S, H)
    out_ref[...] = x3 + csum_ref[...][None] + ttf_ref[...] * dg_ref[...]


def _tc_embed(we3, ttf, Wg, csum, dg, BB=8):
    B, S, E = we3.shape
    H = csum.shape[1]
    return pl.pallas_call(
        _tc_body,
        grid=(B // BB,),
        in_specs=[
            pl.BlockSpec((BB, S, E), lambda i: (i, 0, 0)),
            pl.BlockSpec((BB, S, 1), lambda i: (i, 0, 0)),
            pl.BlockSpec((3 * E, H), lambda i: (0, 0)),
            pl.BlockSpec((S, H), lambda i: (0, 0)),
            pl.BlockSpec((1, H), lambda i: (0, 0)),
        ],
        out_specs=pl.BlockSpec((BB, S, H), lambda i: (i, 0, 0)),
        out_shape=jax.ShapeDtypeStruct((B, S, H), jnp.float32),
    )(we3, ttf, Wg, csum, dg)


def kernel(input_ids, token_type_ids, position_ids, word_table, pos_table,
           type_table, W, b, gamma, beta):
    B, S = input_ids.shape
    V, E = word_table.shape
    H = pos_table.shape[1]

    ids = input_ids.reshape(-1).astype(jnp.int32)
    we = _sc_gather(word_table, ids)

    pe = jnp.take(pos_table, position_ids[0], axis=0)
    Wg = W.T * gamma[None, :]
    csum = (pe + b[None, :] + type_table[0][None, :]) * gamma[None, :] \
        + beta[None, :]
    dg = ((type_table[1] - type_table[0]) * gamma).reshape(1, H)
    ttf = token_type_ids.astype(jnp.float32).reshape(B, S, 1)

    return _tc_embed(we.reshape(B, S, E), ttf, Wg, csum, dg)
